# baseline (speedup 1.0000x reference)
import jax
import jax.numpy as jnp
from jax import lax
from jax.experimental import pallas as pl
from jax.experimental.pallas import tpu as pltpu

WT, WH, WW = 16, 8, 8
NHEADS = 32
NT = 2 * WT - 1
NHW = (2 * WH - 1) * (2 * WW - 1)
Q = WH * WW
QQ = Q * Q
HG = 32


NSPLIT = 8


def _build_g_body(t_ref, d_ref, o_ref):
    oh = (lax.broadcasted_iota(jnp.int32, (NHW, QQ // NSPLIT), 0) == d_ref[...]).astype(
        jnp.float32
    )
    r = jnp.dot(t_ref[...], oh, preferred_element_type=jnp.float32)
    r4 = r.reshape(NHEADS, NT, QQ // NSPLIT // Q, Q)
    o_ref[...] = jnp.concatenate([r4[:, 1:NT], r4[:, 0 : NT - 1]], axis=3)


def _copy_body(g_ref, o_ref):
    i = pl.program_id(1)
    for s in range(WT // 2):
        p = i - 2 * s + WT - 2
        o_ref[:, 0, :, 2 * s * Q : 2 * (s + 1) * Q] = g_ref[:, p]


def kernel(relative_position_bias_table, rel_index):
    table = relative_position_bias_table
    r4 = rel_index.reshape(WT, Q, WT, Q)
    dhw = r4[0, :, WT - 1, :].reshape(1, QQ)

    tableT = (
        table.reshape(NT, NHW, NHEADS).transpose(2, 0, 1).reshape(NHEADS * NT, NHW)
    )

    g = pl.pallas_call(
        _build_g_body,
        grid=(NSPLIT,),
        in_specs=[
            pl.BlockSpec((NHEADS * NT, NHW), lambda n: (0, 0)),
            pl.BlockSpec((1, QQ // NSPLIT), lambda n: (0, n)),
        ],
        out_specs=pl.BlockSpec(
            (NHEADS, NT - 1, Q // NSPLIT, 2 * Q), lambda n: (0, 0, n, 0)
        ),
        out_shape=jax.ShapeDtypeStruct((NHEADS, NT - 1, Q, 2 * Q), jnp.float32),
        compiler_params=pltpu.CompilerParams(
            dimension_semantics=("parallel",)
        ),
    )(tableT, dhw)

    g4 = g

    out4 = pl.pallas_call(
        _copy_body,
        grid=(NHEADS // HG, WT),
        in_specs=[
            pl.BlockSpec((HG, NT - 1, Q, 2 * Q), lambda h, i: (h, 0, 0, 0)),
        ],
        out_specs=pl.BlockSpec((HG, 1, Q, WT * Q), lambda h, i: (h, i, 0, 0)),
        out_shape=jax.ShapeDtypeStruct((NHEADS, WT, Q, WT * Q), jnp.float32),
        compiler_params=pltpu.CompilerParams(
            dimension_semantics=("parallel", "parallel")
        ),
    )(g4)
    return out4.reshape(NHEADS, WT * Q, WT * Q)


# speedup vs baseline: 1.0457x; 1.0457x over previous
import jax
import jax.numpy as jnp
from jax import lax
from jax.experimental import pallas as pl
from jax.experimental.pallas import tpu as pltpu

WT, WH, WW = 16, 8, 8
NHEADS = 32
NT = 2 * WT - 1
NHW = (2 * WH - 1) * (2 * WW - 1)
Q = WH * WW
QQ = Q * Q
HG = 32


NSPLIT = 8


def _build_g_body(t_ref, d_ref, o_ref):
    oh = (lax.broadcasted_iota(jnp.int32, (NHW, QQ // NSPLIT), 0) == d_ref[...]).astype(
        jnp.float32
    )
    r = jnp.dot(t_ref[...], oh, preferred_element_type=jnp.float32)
    o_ref[...] = r.reshape(NHEADS, NT, QQ // NSPLIT // Q, Q)


def _copy_body(g_ref, o_ref):
    i = pl.program_id(1)
    for t2 in range(WT):
        dt = i - t2 + WT - 1
        o_ref[:, 0, :, t2 * Q : (t2 + 1) * Q] = g_ref[:, dt]


def kernel(relative_position_bias_table, rel_index):
    table = relative_position_bias_table
    r4 = rel_index.reshape(WT, Q, WT, Q)
    dhw = r4[0, :, WT - 1, :].reshape(1, QQ)

    tableT = (
        table.reshape(NT, NHW, NHEADS).transpose(2, 0, 1).reshape(NHEADS * NT, NHW)
    )

    g = pl.pallas_call(
        _build_g_body,
        grid=(NSPLIT,),
        in_specs=[
            pl.BlockSpec((NHEADS * NT, NHW), lambda n: (0, 0)),
            pl.BlockSpec((1, QQ // NSPLIT), lambda n: (0, n)),
        ],
        out_specs=pl.BlockSpec((NHEADS, NT, Q // NSPLIT, Q), lambda n: (0, 0, n, 0)),
        out_shape=jax.ShapeDtypeStruct((NHEADS, NT, Q, Q), jnp.float32),
        compiler_params=pltpu.CompilerParams(
            dimension_semantics=("parallel",)
        ),
    )(tableT, dhw)

    g4 = g

    out4 = pl.pallas_call(
        _copy_body,
        grid=(NHEADS // HG, WT),
        in_specs=[
            pl.BlockSpec((HG, NT, Q, Q), lambda h, i: (h, 0, 0, 0)),
        ],
        out_specs=pl.BlockSpec((HG, 1, Q, WT * Q), lambda h, i: (h, i, 0, 0)),
        out_shape=jax.ShapeDtypeStruct((NHEADS, WT, Q, WT * Q), jnp.float32),
        compiler_params=pltpu.CompilerParams(
            dimension_semantics=("parallel", "parallel")
        ),
    )(g4)
    return out4.reshape(NHEADS, WT * Q, WT * Q)
